# baseline (speedup 1.0000x reference)
import jax
import jax.numpy as jnp
from jax.experimental import pallas as pl
from jax.experimental.pallas import tpu as pltpu

_B, _S, _F = 8, 1024, 256
_BPC = 8
_NCH = _B // _BPC


def _body(in_hbm, out_hbm, stage, fillbuf, in_sems, out_sems, fill_sems):
    for c in range(_NCH):
        b = c * _BPC
        pltpu.make_async_copy(
            in_hbm.at[pl.ds(b, _BPC)], stage.at[c], in_sems.at[c]
        ).start()

    minval = None
    for c in range(_NCH):
        b = c * _BPC
        pltpu.make_async_copy(
            in_hbm.at[pl.ds(b, _BPC)], stage.at[c], in_sems.at[c]
        ).wait()
        pltpu.make_async_copy(
            stage.at[c], out_hbm.at[pl.ds(b, _BPC), 0:_S], out_sems.at[c]
        ).start()
        cmin = jnp.min(stage[c])
        minval = cmin if minval is None else jnp.minimum(minval, cmin)

    fillbuf[...] = jnp.full(fillbuf.shape, minval - 1.0, fillbuf.dtype)
    for b in range(_B):
        pltpu.make_async_copy(
            fillbuf, out_hbm.at[b, _S : 2 * _S], fill_sems.at[b]
        ).start()

    for c in range(_NCH):
        b = c * _BPC
        pltpu.make_async_copy(
            stage.at[c], out_hbm.at[pl.ds(b, _BPC), 0:_S], out_sems.at[c]
        ).wait()
    for b in range(_B):
        pltpu.make_async_copy(
            fillbuf, out_hbm.at[b, _S : 2 * _S], fill_sems.at[b]
        ).wait()


def kernel(inputs):
    return pl.pallas_call(
        _body,
        in_specs=[pl.BlockSpec(memory_space=pltpu.MemorySpace.HBM)],
        out_specs=pl.BlockSpec(memory_space=pltpu.MemorySpace.HBM),
        out_shape=jax.ShapeDtypeStruct((_B, 2 * _S, _F), inputs.dtype),
        scratch_shapes=[
            pltpu.VMEM((_NCH, _BPC, _S, _F), jnp.float32),
            pltpu.VMEM((_S, _F), jnp.float32),
            pltpu.SemaphoreType.DMA((_NCH,)),
            pltpu.SemaphoreType.DMA((_NCH,)),
            pltpu.SemaphoreType.DMA((_B,)),
        ],
    )(inputs)


# speedup vs baseline: 1.0876x; 1.0876x over previous
import jax
import jax.numpy as jnp
from jax.experimental import pallas as pl
from jax.experimental.pallas import tpu as pltpu

_B, _S, _F = 8, 1024, 256
_BPC = 4
_NCH = _B // _BPC


def _body(in_hbm, out_hbm, stage, fillbuf, in_sems, out_sems, fill_sems):
    for c in range(_NCH):
        b = c * _BPC
        pltpu.make_async_copy(
            in_hbm.at[pl.ds(b, _BPC)], stage.at[c], in_sems.at[c]
        ).start()

    minval = None
    for c in range(_NCH):
        b = c * _BPC
        pltpu.make_async_copy(
            in_hbm.at[pl.ds(b, _BPC)], stage.at[c], in_sems.at[c]
        ).wait()
        pltpu.make_async_copy(
            stage.at[c], out_hbm.at[pl.ds(b, _BPC), 0:_S], out_sems.at[c]
        ).start()
        cmin = jnp.min(stage[c])
        minval = cmin if minval is None else jnp.minimum(minval, cmin)

    fillbuf[...] = jnp.full(fillbuf.shape, minval - 1.0, fillbuf.dtype)
    for b in range(_B):
        pltpu.make_async_copy(
            fillbuf, out_hbm.at[b, _S : 2 * _S], fill_sems.at[b]
        ).start()

    for c in range(_NCH):
        b = c * _BPC
        pltpu.make_async_copy(
            stage.at[c], out_hbm.at[pl.ds(b, _BPC), 0:_S], out_sems.at[c]
        ).wait()
    for b in range(_B):
        pltpu.make_async_copy(
            fillbuf, out_hbm.at[b, _S : 2 * _S], fill_sems.at[b]
        ).wait()


def kernel(inputs):
    return pl.pallas_call(
        _body,
        in_specs=[pl.BlockSpec(memory_space=pltpu.MemorySpace.HBM)],
        out_specs=pl.BlockSpec(memory_space=pltpu.MemorySpace.HBM),
        out_shape=jax.ShapeDtypeStruct((_B, 2 * _S, _F), inputs.dtype),
        scratch_shapes=[
            pltpu.VMEM((_NCH, _BPC, _S, _F), jnp.float32),
            pltpu.VMEM((_S, _F), jnp.float32),
            pltpu.SemaphoreType.DMA((_NCH,)),
            pltpu.SemaphoreType.DMA((_NCH,)),
            pltpu.SemaphoreType.DMA((_B,)),
        ],
    )(inputs)
